# baseline (speedup 1.0000x reference)
import functools

import jax
import jax.numpy as jnp
from jax import lax
from jax.experimental import pallas as pl
from jax.experimental.pallas import tpu as pltpu
from jax.experimental.pallas import tpu_sc as plsc

NQ = 256
DX = 1024
D = 512
NE = 100000
BLK = 2048
NB = 50
NBLK = 49
NEP = NBLK * BLK
NCHUNK = NBLK * 128
TOPK = 16
BETA = 8.0
NEG = -3.0e38

_HI = jax.lax.Precision.HIGHEST


def _phase_a_body(x_ref, emb_ref, proj_ref, sc_ref, m_ref, xp_s, xn_s):
    i = pl.program_id(0)

    @pl.when(i == 0)
    def _():
        xp = lax.dot_general(x_ref[...], proj_ref[...],
                             (((1,), (1,)), ((), ())),
                             preferred_element_type=jnp.float32,
                             precision=_HI)
        xp_s[...] = xp
        xnsq = jnp.sum(xp * xp, axis=1, keepdims=True)
        xn_s[...] = jnp.broadcast_to(jnp.sqrt(xnsq), (NQ, 128))

    blk = emb_ref[...]
    xp = xp_s[...]
    s = lax.dot_general(xp, blk, (((1,), (1,)), ((), ())),
                        preferred_element_type=jnp.float32, precision=_HI)
    sq = blk * blk
    ensq = lax.dot_general(jnp.ones((8, D), jnp.float32), sq,
                           (((1,), (1,)), ((), ())),
                           preferred_element_type=jnp.float32,
                           precision=_HI)[0:1, :]
    en = jnp.sqrt(ensq)
    xn = xn_s[:, 0:1]
    scores = s / (xn * en + 1e-8)
    col = BLK * i + lax.broadcasted_iota(jnp.int32, (NQ, BLK), 1)
    scores = jnp.where(col < NE, scores, NEG)
    sc_ref[...] = scores
    m_ref[...] = jnp.max(scores.reshape(NQ, 16, 128), axis=1)[None]


def _phase_a(xq, embeddings, proj):
    return pl.pallas_call(
        _phase_a_body,
        grid=(NBLK,),
        in_specs=[
            pl.BlockSpec((NQ, DX), lambda i: (0, 0)),
            pl.BlockSpec((BLK, D), lambda i: (i, 0)),
            pl.BlockSpec((D, DX), lambda i: (0, 0)),
        ],
        out_specs=[
            pl.BlockSpec((NQ, BLK), lambda i: (0, i)),
            pl.BlockSpec((1, NQ, 128), lambda i: (i, 0, 0)),
        ],
        out_shape=[
            jax.ShapeDtypeStruct((NQ, NEP), jnp.float32),
            jax.ShapeDtypeStruct((NBLK, NQ, 128), jnp.float32),
        ],
        scratch_shapes=[
            pltpu.VMEM((NQ, D), jnp.float32),
            pltpu.VMEM((NQ, 128), jnp.float32),
        ],
        compiler_params=pltpu.CompilerParams(
            dimension_semantics=("arbitrary",)),
    )(xq, embeddings, proj)


def _phase_b_body(m_ref, ccol_ref, sidx_ref):
    mv = m_ref[...]
    b_iota = lax.broadcasted_iota(jnp.int32, (NBLK, NQ, 128), 0)
    l_iota = lax.broadcasted_iota(jnp.int32, (NBLK, NQ, 128), 2)
    cbase = b_iota * BLK + l_iota
    big = jnp.int32(2 ** 30)
    sels = []
    for _ in range(TOPK):
        m1 = jnp.max(mv, axis=0)
        mm = jnp.max(m1, axis=1, keepdims=True)
        eq = mv == mm[None, :, :]
        t = jnp.min(jnp.where(eq, cbase, big), axis=0)
        sel = jnp.min(t, axis=1, keepdims=True)
        sels.append(sel)
        mv = jnp.where(cbase == sel[None, :, :], NEG, mv)
    selmat = jnp.concatenate(sels, axis=1)
    j_iota = lax.broadcasted_iota(jnp.int32, (NQ, TOPK, 16), 2)
    ccol = selmat[:, :, None] + 128 * j_iota
    q_iota = lax.broadcasted_iota(jnp.int32, (NQ, TOPK, 16), 0)
    ccol_ref[...] = ccol
    sidx_ref[...] = ccol + q_iota * NEP


def _phase_b(m):
    return pl.pallas_call(
        _phase_b_body,
        out_shape=[
            jax.ShapeDtypeStruct((NQ, TOPK, 16), jnp.int32),
            jax.ShapeDtypeStruct((NQ, TOPK, 16), jnp.int32),
        ],
    )(m)


def _phase_d_body(cand_ref, col_ref, w_ref, id_ref):
    cv = cand_ref[...]
    cols = col_ref[...]
    big = jnp.int32(2 ** 30)
    vals, ids = [], []
    for _ in range(TOPK):
        m = jnp.max(cv, axis=1, keepdims=True)
        sel = jnp.min(jnp.where(cv == m, cols, big), axis=1, keepdims=True)
        vals.append(m)
        ids.append(sel)
        cv = jnp.where(cols == sel, NEG, cv)
    v = jnp.concatenate(vals, axis=1)
    ind = jnp.concatenate(ids, axis=1)
    z = BETA * v
    z = z - jnp.max(z, axis=1, keepdims=True)
    e = jnp.exp(z)
    w = e / jnp.sum(e, axis=1, keepdims=True)
    w_ref[...] = jnp.broadcast_to(w[:, :, None], (NQ, TOPK, 16))
    id_ref[...] = ind


def _phase_d(cand, ccol):
    return pl.pallas_call(
        _phase_d_body,
        out_shape=[
            jax.ShapeDtypeStruct((NQ, TOPK, 16), jnp.float32),
            jax.ShapeDtypeStruct((NQ, TOPK), jnp.int32),
        ],
    )(cand, ccol)


_NC = 2
_NS = 16
_NW = _NC * _NS

_C_PER_W = (NQ * 256) // _NW
_E_PER_W = (NQ * TOPK) // _NW
_QB = NQ // _NW


def _phase_c_body(table, idx_hbm, out_hbm, idx_v, vals_v, sem):
    wid = lax.axis_index("s") * _NC + lax.axis_index("c")
    base = wid * _C_PER_W
    pltpu.sync_copy(idx_hbm.at[pl.ds(base, _C_PER_W)], idx_v)
    copies = []
    for t in range(_C_PER_W // 128):
        copies.append(pltpu.async_copy(
            table.at[idx_v.at[pl.ds(t * 128, 128)]],
            vals_v.at[pl.ds(t * 128, 128), :], sem))
    for cp in copies:
        cp.wait()
    pltpu.sync_copy(vals_v, out_hbm.at[pl.ds(base, _C_PER_W), :])


def _phase_c(scores_flat, sidx_flat):
    mesh = plsc.VectorSubcoreMesh(core_axis_name="c", subcore_axis_name="s")
    kfn = functools.partial(
        pl.kernel,
        mesh=mesh,
        out_type=jax.ShapeDtypeStruct((NQ * 256, 1), jnp.float32),
        scratch_types=[
            pltpu.VMEM((_C_PER_W,), jnp.int32),
            pltpu.VMEM((_C_PER_W, 1), jnp.float32),
            pltpu.SemaphoreType.DMA,
        ],
        compiler_params=pltpu.CompilerParams(use_tc_tiling_on_sc=False),
    )(_phase_c_body)
    return kfn(scores_flat, sidx_flat)


def _phase_e_body(table, idx_hbm, w_hbm, out_hbm, idx_v, w_v, rows_v, acc_v,
                  sem):
    wid = lax.axis_index("s") * _NC + lax.axis_index("c")
    base = wid * _E_PER_W
    pltpu.sync_copy(idx_hbm.at[pl.ds(base, _E_PER_W)], idx_v)
    pltpu.sync_copy(w_hbm.at[pl.ds(base, _E_PER_W), :], w_v)
    pltpu.async_copy(table.at[idx_v], rows_v, sem).wait()

    def q_body(q, _):
        def v_body(v, __):
            acc = w_v[q * TOPK, :] * rows_v[q * TOPK, pl.ds(v * 16, 16)]
            for k in range(1, TOPK):
                acc = acc + (w_v[q * TOPK + k, :]
                             * rows_v[q * TOPK + k, pl.ds(v * 16, 16)])
            acc_v[q, pl.ds(v * 16, 16)] = acc
            return __

        return lax.fori_loop(0, D // 16, v_body, _)

    lax.fori_loop(0, _QB, q_body, None)
    pltpu.sync_copy(acc_v, out_hbm.at[pl.ds(wid * _QB, _QB), :])


def _phase_e(embeddings, ids_flat, w_flat):
    mesh = plsc.VectorSubcoreMesh(core_axis_name="c", subcore_axis_name="s")
    kfn = functools.partial(
        pl.kernel,
        mesh=mesh,
        out_type=jax.ShapeDtypeStruct((NQ, D), jnp.float32),
        scratch_types=[
            pltpu.VMEM((_E_PER_W,), jnp.int32),
            pltpu.VMEM((_E_PER_W, 16), jnp.float32),
            pltpu.VMEM((_E_PER_W, D), jnp.float32),
            pltpu.VMEM((_QB, D), jnp.float32),
            pltpu.SemaphoreType.DMA,
        ],
    )(_phase_e_body)
    return kfn(embeddings, ids_flat, w_flat)


def kernel(x, embeddings, proj):
    xq = x.reshape(NQ, DX)
    scores, m = _phase_a(xq, embeddings, proj)
    ccol, sidx = _phase_b(m)
    cand = _phase_c(scores.reshape(NQ * NEP, 1), sidx.reshape(NQ * 256))
    w_exp, ids = _phase_d(cand.reshape(NQ, 256), ccol.reshape(NQ, 256))
    out = _phase_e(embeddings, ids.reshape(NQ * TOPK),
                   w_exp.reshape(NQ * TOPK, 16))
    return out.reshape(32, 8, D)
